# baseline (speedup 1.0000x reference)
import functools

import jax
import jax.numpy as jnp
from jax import lax
from jax.experimental import pallas as pl
from jax.experimental.pallas import tpu as pltpu
from jax.experimental.pallas import tpu_sc as plsc

N = 50000
E = 800000
NODE_F = 4
EDGE_F = 16
HID = 128
OUT1 = 28
OUTP = 32
EPS = 1e-5

NC = 2
NS = 16
NW = NC * NS
CHUNK = 128
NCHUNKS = E // CHUNK
BASE_CH = NCHUNKS // NW
EXTRA = NCHUNKS - BASE_CH * NW
ROWS_PER_TILE = 3136
NPAD = NS * ROWS_PER_TILE

BLK_A = 8000
BLK_B = 6400
BLK_N = 5000

_f32 = jnp.float32


def _moments_body(ea_ref, g_ref, s_ref, pk_ref, acc_g, acc_s):
    i = pl.program_id(0)

    @pl.when(i == 0)
    def _init():
        acc_g[...] = jnp.zeros_like(acc_g)
        acc_s[...] = jnp.zeros_like(acc_s)

    blk = ea_ref[...]
    acc_g[...] += lax.dot_general(blk, blk, (((0,), (0,)), ((), ())),
                                  preferred_element_type=_f32)
    acc_s[...] += jnp.sum(blk, axis=0, keepdims=True)
    blk3 = blk.reshape(BLK_A // 8, 8, EDGE_F)
    pk_ref[...] = jnp.concatenate(
        [blk3[:, j, :] for j in range(8)], axis=1)

    @pl.when(i == pl.num_programs(0) - 1)
    def _emit():
        g_ref[...] = acc_g[...]
        s_ref[...] = acc_s[...]


def _moments(edge_attr):
    return pl.pallas_call(
        _moments_body,
        grid=(E // BLK_A,),
        in_specs=[pl.BlockSpec((BLK_A, EDGE_F), lambda i: (i, 0))],
        out_specs=[pl.BlockSpec((EDGE_F, EDGE_F), lambda i: (0, 0)),
                   pl.BlockSpec((1, EDGE_F), lambda i: (0, 0)),
                   pl.BlockSpec((BLK_A // 8, 128), lambda i: (i, 0))],
        out_shape=[jax.ShapeDtypeStruct((EDGE_F, EDGE_F), _f32),
                   jax.ShapeDtypeStruct((1, EDGE_F), _f32),
                   jax.ShapeDtypeStruct((E // 8, 128), _f32)],
        scratch_shapes=[pltpu.VMEM((EDGE_F, EDGE_F), _f32),
                        pltpu.VMEM((1, EDGE_F), _f32)],
    )(edge_attr)


def _worker_partition(wid):
    n = BASE_CH + jnp.where(wid < EXTRA, 1, 0)
    start = wid * BASE_CH + jnp.minimum(wid, EXTRA)
    return start, n


SUP = 8


def _sc_gather(x_pad, src):
    mesh = plsc.VectorSubcoreMesh(core_axis_name="c", subcore_axis_name="s")

    @functools.partial(
        pl.kernel,
        out_type=jax.ShapeDtypeStruct((E, 8), _f32),
        mesh=mesh,
        compiler_params=pltpu.CompilerParams(use_tc_tiling_on_sc=False),
        scratch_types=[pltpu.VMEM((SUP * CHUNK,), jnp.int32),
                       pltpu.VMEM((SUP * CHUNK, 8), _f32),
                       pltpu.SemaphoreType.DMA],
    )
    def gk(x_hbm, src_hbm, out_hbm, idx_v, rows_v, sem):
        wid = lax.axis_index("s") * NC + lax.axis_index("c")
        start, n = _worker_partition(wid)

        def super_body(t, carry):
            off = (start + t * SUP) * CHUNK
            pltpu.sync_copy(src_hbm.at[pl.ds(off, SUP * CHUNK)], idx_v)
            copies = [
                pltpu.async_copy(x_hbm.at[idx_v.at[pl.ds(k * CHUNK, CHUNK)]],
                                 rows_v.at[pl.ds(k * CHUNK, CHUNK)], sem)
                for k in range(SUP)]
            for c in copies:
                c.wait()
            pltpu.sync_copy(rows_v, out_hbm.at[pl.ds(off, SUP * CHUNK)])
            return carry

        def tail_body(j, carry):
            off = (start + j) * CHUNK
            pltpu.sync_copy(src_hbm.at[pl.ds(off, CHUNK)],
                            idx_v.at[pl.ds(0, CHUNK)])
            pltpu.async_copy(x_hbm.at[idx_v.at[pl.ds(0, CHUNK)]],
                             rows_v.at[pl.ds(0, CHUNK)], sem).wait()
            pltpu.sync_copy(rows_v.at[pl.ds(0, CHUNK)],
                            out_hbm.at[pl.ds(off, CHUNK)])
            return carry

        nsup = n // SUP
        lax.fori_loop(0, nsup, super_body, 0)
        lax.fori_loop(nsup * SUP, n, tail_body, 0)

    return gk(x_pad, src)


def _edge_body(ea_ref, xj_ref, w1_ref, c1_ref, w2_ref, b2_ref, msg_ref):
    eap = ea_ref[...]
    ea = jnp.concatenate(
        [eap[:, 16 * j:16 * (j + 1)].reshape(BLK_B // 8, 1, EDGE_F)
         for j in range(8)], axis=1).reshape(BLK_B, EDGE_F)
    h1 = jnp.maximum(
        jnp.dot(ea, w1_ref[...], preferred_element_type=_f32)
        + c1_ref[...], 0.0)
    h2 = jnp.maximum(
        jnp.dot(h1, w2_ref[...], preferred_element_type=_f32)
        + b2_ref[...], 0.0)
    xjp = xj_ref[...]
    xj = jnp.concatenate(
        [xjp[:, 8 * m:8 * (m + 1)].reshape(BLK_B // 16, 1, 8)
         for m in range(16)], axis=1).reshape(BLK_B, 8)
    acc = xj[:, 0:1] * h2[:, 0:OUTP]
    for i in (1, 2, 3):
        acc = acc + xj[:, i:i + 1] * h2[:, OUTP * i:OUTP * (i + 1)]
    acc3 = acc.reshape(BLK_B // 4, 4, OUTP)
    msg_ref[...] = jnp.concatenate(
        [acc3[:, u, :] for u in range(4)], axis=1)


def _edge_msgs(ea_packed, xj_packed, w1f, c1, w2p, b2p):
    return pl.pallas_call(
        _edge_body,
        grid=(E // BLK_B,),
        in_specs=[pl.BlockSpec((BLK_B // 8, 128), lambda i: (i, 0)),
                  pl.BlockSpec((BLK_B // 16, 128), lambda i: (i, 0)),
                  pl.BlockSpec((EDGE_F, HID), lambda i: (0, 0)),
                  pl.BlockSpec((1, HID), lambda i: (0, 0)),
                  pl.BlockSpec((HID, HID), lambda i: (0, 0)),
                  pl.BlockSpec((1, HID), lambda i: (0, 0))],
        out_specs=pl.BlockSpec((BLK_B // 4, 128), lambda i: (i, 0)),
        out_shape=jax.ShapeDtypeStruct((E // 4, 128), _f32),
    )(ea_packed, xj_packed, w1f, c1, w2p, b2p)


H_HALF = NPAD // 2
ROWS_T = H_HALF // NS
ACC_ROWS = H_HALF + NS
CH_BASE = NCHUNKS // NS
CH_EXTRA = NCHUNKS - CH_BASE * NS


def _sc_scatter(msg, dst, zrows):
    mesh = plsc.VectorSubcoreMesh(core_axis_name="c", subcore_axis_name="s")

    @functools.partial(
        pl.kernel,
        out_type=jax.ShapeDtypeStruct((NPAD, OUTP), _f32),
        mesh=mesh,
        compiler_params=pltpu.CompilerParams(use_tc_tiling_on_sc=False),
        scratch_types=[pltpu.VMEM((SUP * CHUNK,), jnp.int32),
                       pltpu.VMEM((SUP, CHUNK), jnp.int32),
                       pltpu.VMEM((SUP * CHUNK, OUTP), _f32),
                       pltpu.VMEM_SHARED((ACC_ROWS, OUTP), _f32),
                       pltpu.SemaphoreType.DMA],
    )
    def sk(msg_hbm, dst_hbm, z_hbm, out_hbm, idx_v, loc_v, rows_v,
           accum, sem):
        cid = lax.axis_index("c")
        sid = lax.axis_index("s")
        base_node = cid * H_HALF
        pltpu.sync_copy(z_hbm, accum.at[pl.ds(sid * ROWS_T, ROWS_T)])
        plsc.subcore_barrier()

        n = CH_BASE + jnp.where(sid < CH_EXTRA, 1, 0)
        start = sid * CH_BASE + jnp.minimum(sid, CH_EXTRA)

        def filter_chunk(k):
            for m in range(CHUNK // 16):
                v = idx_v[pl.ds(k * CHUNK + m * 16, 16)] - base_node
                ok = (v >= 0) & (v < H_HALF)
                loc_v[k, pl.ds(m * 16, 16)] = jnp.where(
                    ok, v, jnp.int32(H_HALF))

        def super_body(t, carry):
            off = (start + t * SUP) * CHUNK
            pltpu.sync_copy(dst_hbm.at[pl.ds(off, SUP * CHUNK)], idx_v)
            pltpu.sync_copy(msg_hbm.at[pl.ds(off, SUP * CHUNK)], rows_v)
            copies = []
            for k in range(SUP):
                filter_chunk(k)
                copies.append(pltpu.async_copy(
                    rows_v.at[pl.ds(k * CHUNK, CHUNK)],
                    accum.at[loc_v.at[k]], sem, add=True))
            for c in copies:
                c.wait()
            return carry

        def tail_body(j, carry):
            off = (start + j) * CHUNK
            pltpu.sync_copy(dst_hbm.at[pl.ds(off, CHUNK)],
                            idx_v.at[pl.ds(0, CHUNK)])
            pltpu.sync_copy(msg_hbm.at[pl.ds(off, CHUNK)],
                            rows_v.at[pl.ds(0, CHUNK)])
            filter_chunk(0)
            pltpu.sync_copy(rows_v.at[pl.ds(0, CHUNK)],
                            accum.at[loc_v.at[0]], add=True)
            return carry

        nsup = n // SUP
        lax.fori_loop(0, nsup, super_body, 0)
        lax.fori_loop(nsup * SUP, n, tail_body, 0)
        plsc.subcore_barrier()
        pltpu.sync_copy(accum.at[pl.ds(sid * ROWS_T, ROWS_T)],
                        out_hbm.at[pl.ds(base_node + sid * ROWS_T, ROWS_T)])

    return sk(msg, dst, zrows)


def _c1_body(p_ref, x_ref, r_ref, b_ref, out_ref, st_ref, acc):
    i = pl.program_id(0)

    @pl.when(i == 0)
    def _init():
        acc[...] = jnp.zeros_like(acc)

    blk = (p_ref[...]
           + jnp.dot(x_ref[...], r_ref[...], preferred_element_type=_f32)
           + b_ref[...])
    out_ref[...] = blk
    acc[0:1, :] += jnp.sum(blk, axis=0, keepdims=True)
    acc[1:2, :] += jnp.sum(blk * blk, axis=0, keepdims=True)

    @pl.when(i == pl.num_programs(0) - 1)
    def _emit():
        st_ref[...] = acc[...]


def _combine(p, x, root32, bias32):
    return pl.pallas_call(
        _c1_body,
        grid=(N // BLK_N,),
        in_specs=[pl.BlockSpec((BLK_N, OUTP), lambda i: (i, 0)),
                  pl.BlockSpec((BLK_N, NODE_F), lambda i: (i, 0)),
                  pl.BlockSpec((NODE_F, OUTP), lambda i: (0, 0)),
                  pl.BlockSpec((1, OUTP), lambda i: (0, 0))],
        out_specs=[pl.BlockSpec((BLK_N, OUTP), lambda i: (i, 0)),
                   pl.BlockSpec((2, OUTP), lambda i: (0, 0))],
        out_shape=[jax.ShapeDtypeStruct((N, OUTP), _f32),
                   jax.ShapeDtypeStruct((2, OUTP), _f32)],
        scratch_shapes=[pltpu.VMEM((2, OUTP), _f32)],
    )(p, x, root32, bias32)


def _c2_body(o_ref, inv_ref, sh_ref, y_ref):
    y_ref[...] = jnp.maximum(o_ref[...] * inv_ref[...] + sh_ref[...], 0.0)


def _epilogue(out32, inv32, sh32):
    return pl.pallas_call(
        _c2_body,
        grid=(N // BLK_N,),
        in_specs=[pl.BlockSpec((BLK_N, OUTP), lambda i: (i, 0)),
                  pl.BlockSpec((1, OUTP), lambda i: (0, 0)),
                  pl.BlockSpec((1, OUTP), lambda i: (0, 0))],
        out_specs=pl.BlockSpec((BLK_N, OUTP), lambda i: (i, 0)),
        out_shape=jax.ShapeDtypeStruct((N, OUTP), _f32),
    )(out32, inv32, sh32)


def kernel(x, edge_index, edge_attr, W1, b1, bn1_g, bn1_b, W2, b2,
           root, bias, bn_g, bn_b):
    src = edge_index[0]
    dst = edge_index[1]

    G, s, ea_packed = _moments(edge_attr)
    mu = s[0] / E
    cov = G / E - jnp.outer(mu, mu)
    var_h = jnp.sum(W1 * (cov @ W1), axis=0)
    mean_h = mu @ W1 + b1
    scale = bn1_g / jnp.sqrt(var_h + EPS)
    w1f = W1 * scale[None, :]
    c1 = (b1 - mean_h) * scale + bn1_b
    c1 = c1.reshape(1, HID)

    w2g = W2.reshape(HID, NODE_F, OUT1)
    w2p = jnp.zeros((HID, NODE_F, OUTP), _f32).at[:, :, :OUT1].set(w2g)
    w2p = w2p.reshape(HID, NODE_F * OUTP)
    b2g = b2.reshape(NODE_F, OUT1)
    b2p = jnp.zeros((NODE_F, OUTP), _f32).at[:, :OUT1].set(b2g)
    b2p = b2p.reshape(1, NODE_F * OUTP)

    x_pad = jnp.concatenate([x, jnp.zeros((N, 8 - NODE_F), _f32)], axis=1)
    x_j = _sc_gather(x_pad, src)

    xj_packed = jnp.reshape(x_j, (E // 16, 128))
    msg_packed = _edge_msgs(ea_packed, xj_packed, w1f, c1, w2p, b2p)

    msg = jnp.reshape(msg_packed, (E, OUTP))
    zrows = jnp.zeros((ROWS_T, OUTP), _f32)
    aggr = _sc_scatter(msg, dst, zrows)

    root32 = jnp.concatenate([root, jnp.eye(NODE_F, dtype=_f32)], axis=1)
    bias32 = jnp.concatenate([bias, jnp.zeros((NODE_F,), _f32)])
    bias32 = bias32.reshape(1, OUTP)
    out32, st = _combine(aggr, x, root32, bias32)

    mean = st[0] / N
    var = st[1] / N - mean * mean
    inv = jnp.concatenate([bn_g, jnp.ones((NODE_F,), _f32)])
    inv = inv / jnp.sqrt(jnp.concatenate(
        [var[:OUT1], jnp.ones((NODE_F,), _f32)]) + jnp.concatenate(
        [jnp.full((OUT1,), EPS, _f32), jnp.zeros((NODE_F,), _f32)]))
    sh = jnp.concatenate([bn_b, jnp.zeros((NODE_F,), _f32)]) - mean * inv
    sh = sh.at[OUT1:].set(0.0)

    return _epilogue(out32, inv.reshape(1, OUTP), sh.reshape(1, OUTP))


# speedup vs baseline: 1.9624x; 1.9624x over previous
import functools

import jax
import jax.numpy as jnp
import numpy as np
from jax import lax
from jax.experimental import pallas as pl
from jax.experimental.pallas import tpu as pltpu
from jax.experimental.pallas import tpu_sc as plsc

N = 50000
E = 800000
NODE_F = 4
EDGE_F = 16
HID = 128
OUT1 = 28
OUTP = 32
EPS = 1e-5

NC = 2
NS = 16
NW = NC * NS
CHUNK = 128
NCHUNKS = E // CHUNK
BASE_CH = NCHUNKS // NW
EXTRA = NCHUNKS - BASE_CH * NW
ROWS_PER_TILE = 3136
NPAD = NS * ROWS_PER_TILE

BLK_A = 8000
BLK_B = 6400
BLK_N = 5000

_f32 = jnp.float32

_XB = np.zeros((128, 16 * 128), np.float32)
for _m in range(16):
    for _i in range(NODE_F):
        _XB[8 * _m + _i, 128 * _m + 32 * _i:128 * _m + 32 * (_i + 1)] = 1.0


def _moments_body(ea_ref, g_ref, s_ref, pk_ref, acc_g, acc_s):
    i = pl.program_id(0)

    @pl.when(i == 0)
    def _init():
        acc_g[...] = jnp.zeros_like(acc_g)
        acc_s[...] = jnp.zeros_like(acc_s)

    blk = ea_ref[...]
    acc_g[...] += lax.dot_general(blk, blk, (((0,), (0,)), ((), ())),
                                  preferred_element_type=_f32)
    acc_s[...] += jnp.sum(blk, axis=0, keepdims=True)
    blk3 = blk.reshape(BLK_A // 8, 8, EDGE_F)
    pk_ref[...] = jnp.concatenate(
        [blk3[:, j, :] for j in range(8)], axis=1)

    @pl.when(i == pl.num_programs(0) - 1)
    def _emit():
        g_ref[...] = acc_g[...]
        s_ref[...] = acc_s[...]


def _moments(edge_attr):
    return pl.pallas_call(
        _moments_body,
        grid=(E // BLK_A,),
        in_specs=[pl.BlockSpec((BLK_A, EDGE_F), lambda i: (i, 0))],
        out_specs=[pl.BlockSpec((EDGE_F, EDGE_F), lambda i: (0, 0)),
                   pl.BlockSpec((1, EDGE_F), lambda i: (0, 0)),
                   pl.BlockSpec((BLK_A // 8, 128), lambda i: (i, 0))],
        out_shape=[jax.ShapeDtypeStruct((EDGE_F, EDGE_F), _f32),
                   jax.ShapeDtypeStruct((1, EDGE_F), _f32),
                   jax.ShapeDtypeStruct((E // 8, 128), _f32)],
        scratch_shapes=[pltpu.VMEM((EDGE_F, EDGE_F), _f32),
                        pltpu.VMEM((1, EDGE_F), _f32)],
    )(edge_attr)


def _worker_partition(wid):
    n = BASE_CH + jnp.where(wid < EXTRA, 1, 0)
    start = wid * BASE_CH + jnp.minimum(wid, EXTRA)
    return start, n


SUP = 8


def _sc_gather(x_pad, src):
    mesh = plsc.VectorSubcoreMesh(core_axis_name="c", subcore_axis_name="s")

    @functools.partial(
        pl.kernel,
        out_type=jax.ShapeDtypeStruct((E, 8), _f32),
        mesh=mesh,
        compiler_params=pltpu.CompilerParams(use_tc_tiling_on_sc=False),
        scratch_types=[pltpu.VMEM((SUP * CHUNK,), jnp.int32),
                       pltpu.VMEM((SUP * CHUNK, 8), _f32),
                       pltpu.SemaphoreType.DMA],
    )
    def gk(x_hbm, src_hbm, out_hbm, idx_v, rows_v, sem):
        wid = lax.axis_index("s") * NC + lax.axis_index("c")
        start, n = _worker_partition(wid)

        def super_body(t, carry):
            off = (start + t * SUP) * CHUNK
            pltpu.sync_copy(src_hbm.at[pl.ds(off, SUP * CHUNK)], idx_v)
            copies = [
                pltpu.async_copy(x_hbm.at[idx_v.at[pl.ds(k * CHUNK, CHUNK)]],
                                 rows_v.at[pl.ds(k * CHUNK, CHUNK)], sem)
                for k in range(SUP)]
            for c in copies:
                c.wait()
            pltpu.sync_copy(rows_v, out_hbm.at[pl.ds(off, SUP * CHUNK)])
            return carry

        def tail_body(j, carry):
            off = (start + j) * CHUNK
            pltpu.sync_copy(src_hbm.at[pl.ds(off, CHUNK)],
                            idx_v.at[pl.ds(0, CHUNK)])
            pltpu.async_copy(x_hbm.at[idx_v.at[pl.ds(0, CHUNK)]],
                             rows_v.at[pl.ds(0, CHUNK)], sem).wait()
            pltpu.sync_copy(rows_v.at[pl.ds(0, CHUNK)],
                            out_hbm.at[pl.ds(off, CHUNK)])
            return carry

        nsup = n // SUP
        lax.fori_loop(0, nsup, super_body, 0)
        lax.fori_loop(nsup * SUP, n, tail_body, 0)

    return gk(x_pad, src)


def _edge_body(ea_ref, xj_ref, w1_ref, c1_ref, w2_ref, b2_ref, xb_ref,
               msg_ref):
    h1p = jnp.maximum(
        jnp.dot(ea_ref[...], w1_ref[...], preferred_element_type=_f32)
        + c1_ref[...], 0.0)
    h1 = h1p.reshape(BLK_B, HID)
    h2 = jnp.maximum(
        jnp.dot(h1, w2_ref[...], preferred_element_type=_f32)
        + b2_ref[...], 0.0)
    xbc = jnp.dot(xj_ref[...], xb_ref[...],
                  preferred_element_type=_f32).reshape(BLK_B, 128)
    t = xbc * h2
    m = (t[:, 0:OUTP] + t[:, OUTP:2 * OUTP]
         + t[:, 2 * OUTP:3 * OUTP] + t[:, 3 * OUTP:4 * OUTP])
    m3 = m.reshape(BLK_B // 4, 4, OUTP)
    msg_ref[...] = jnp.concatenate(
        [m3[:, u, :] for u in range(4)], axis=1)


def _edge_msgs(ea_packed, xj_packed, w1big, c1big, w2p, b2p, xbig):
    return pl.pallas_call(
        _edge_body,
        grid=(E // BLK_B,),
        in_specs=[pl.BlockSpec((BLK_B // 8, 128), lambda i: (i, 0)),
                  pl.BlockSpec((BLK_B // 16, 128), lambda i: (i, 0)),
                  pl.BlockSpec((HID, 8 * HID), lambda i: (0, 0)),
                  pl.BlockSpec((1, 8 * HID), lambda i: (0, 0)),
                  pl.BlockSpec((HID, HID), lambda i: (0, 0)),
                  pl.BlockSpec((1, HID), lambda i: (0, 0)),
                  pl.BlockSpec((128, 16 * 128), lambda i: (0, 0))],
        out_specs=pl.BlockSpec((BLK_B // 4, 128), lambda i: (i, 0)),
        out_shape=jax.ShapeDtypeStruct((E // 4, 128), _f32),
    )(ea_packed, xj_packed, w1big, c1big, w2p, b2p, xbig)


H_HALF = NPAD // 2
ROWS_T = H_HALF // NS
ACC_ROWS = H_HALF + NS
CH_BASE = NCHUNKS // NS
CH_EXTRA = NCHUNKS - CH_BASE * NS


def _sc_scatter(msg, dst, zrows):
    mesh = plsc.VectorSubcoreMesh(core_axis_name="c", subcore_axis_name="s")

    @functools.partial(
        pl.kernel,
        out_type=jax.ShapeDtypeStruct((NPAD, OUTP), _f32),
        mesh=mesh,
        compiler_params=pltpu.CompilerParams(use_tc_tiling_on_sc=False),
        scratch_types=[pltpu.VMEM((SUP * CHUNK,), jnp.int32),
                       pltpu.VMEM((SUP, CHUNK), jnp.int32),
                       pltpu.VMEM((SUP * CHUNK, OUTP), _f32),
                       pltpu.VMEM_SHARED((ACC_ROWS, OUTP), _f32),
                       pltpu.SemaphoreType.DMA],
    )
    def sk(msg_hbm, dst_hbm, z_hbm, out_hbm, idx_v, loc_v, rows_v,
           accum, sem):
        cid = lax.axis_index("c")
        sid = lax.axis_index("s")
        base_node = cid * H_HALF
        pltpu.sync_copy(z_hbm, accum.at[pl.ds(sid * ROWS_T, ROWS_T)])
        plsc.subcore_barrier()

        n = CH_BASE + jnp.where(sid < CH_EXTRA, 1, 0)
        start = sid * CH_BASE + jnp.minimum(sid, CH_EXTRA)

        def filter_chunk(k):
            for m in range(CHUNK // 16):
                v = idx_v[pl.ds(k * CHUNK + m * 16, 16)] - base_node
                ok = (v >= 0) & (v < H_HALF)
                loc_v[k, pl.ds(m * 16, 16)] = jnp.where(
                    ok, v, jnp.int32(H_HALF))

        def super_body(t, carry):
            off = (start + t * SUP) * CHUNK
            pltpu.sync_copy(dst_hbm.at[pl.ds(off, SUP * CHUNK)], idx_v)
            pltpu.sync_copy(msg_hbm.at[pl.ds(off, SUP * CHUNK)], rows_v)
            copies = []
            for k in range(SUP):
                filter_chunk(k)
                copies.append(pltpu.async_copy(
                    rows_v.at[pl.ds(k * CHUNK, CHUNK)],
                    accum.at[loc_v.at[k]], sem, add=True))
            for c in copies:
                c.wait()
            return carry

        def tail_body(j, carry):
            off = (start + j) * CHUNK
            pltpu.sync_copy(dst_hbm.at[pl.ds(off, CHUNK)],
                            idx_v.at[pl.ds(0, CHUNK)])
            pltpu.sync_copy(msg_hbm.at[pl.ds(off, CHUNK)],
                            rows_v.at[pl.ds(0, CHUNK)])
            filter_chunk(0)
            pltpu.sync_copy(rows_v.at[pl.ds(0, CHUNK)],
                            accum.at[loc_v.at[0]], add=True)
            return carry

        nsup = n // SUP
        lax.fori_loop(0, nsup, super_body, 0)
        lax.fori_loop(nsup * SUP, n, tail_body, 0)
        plsc.subcore_barrier()
        pltpu.sync_copy(accum.at[pl.ds(sid * ROWS_T, ROWS_T)],
                        out_hbm.at[pl.ds(base_node + sid * ROWS_T, ROWS_T)])

    return sk(msg, dst, zrows)


def _c1_body(p_ref, x_ref, r_ref, b_ref, out_ref, st_ref, acc):
    i = pl.program_id(0)

    @pl.when(i == 0)
    def _init():
        acc[...] = jnp.zeros_like(acc)

    blk = (p_ref[...]
           + jnp.dot(x_ref[...], r_ref[...], preferred_element_type=_f32)
           + b_ref[...])
    out_ref[...] = blk
    acc[0:1, :] += jnp.sum(blk, axis=0, keepdims=True)
    acc[1:2, :] += jnp.sum(blk * blk, axis=0, keepdims=True)

    @pl.when(i == pl.num_programs(0) - 1)
    def _emit():
        st_ref[...] = acc[...]


def _combine(p, x, root32, bias32):
    return pl.pallas_call(
        _c1_body,
        grid=(N // BLK_N,),
        in_specs=[pl.BlockSpec((BLK_N, OUTP), lambda i: (i, 0)),
                  pl.BlockSpec((BLK_N, NODE_F), lambda i: (i, 0)),
                  pl.BlockSpec((NODE_F, OUTP), lambda i: (0, 0)),
                  pl.BlockSpec((1, OUTP), lambda i: (0, 0))],
        out_specs=[pl.BlockSpec((BLK_N, OUTP), lambda i: (i, 0)),
                   pl.BlockSpec((2, OUTP), lambda i: (0, 0))],
        out_shape=[jax.ShapeDtypeStruct((N, OUTP), _f32),
                   jax.ShapeDtypeStruct((2, OUTP), _f32)],
        scratch_shapes=[pltpu.VMEM((2, OUTP), _f32)],
    )(p, x, root32, bias32)


def _c2_body(o_ref, inv_ref, sh_ref, y_ref):
    y_ref[...] = jnp.maximum(o_ref[...] * inv_ref[...] + sh_ref[...], 0.0)


def _epilogue(out32, inv32, sh32):
    return pl.pallas_call(
        _c2_body,
        grid=(N // BLK_N,),
        in_specs=[pl.BlockSpec((BLK_N, OUTP), lambda i: (i, 0)),
                  pl.BlockSpec((1, OUTP), lambda i: (0, 0)),
                  pl.BlockSpec((1, OUTP), lambda i: (0, 0))],
        out_specs=pl.BlockSpec((BLK_N, OUTP), lambda i: (i, 0)),
        out_shape=jax.ShapeDtypeStruct((N, OUTP), _f32),
    )(out32, inv32, sh32)


def kernel(x, edge_index, edge_attr, W1, b1, bn1_g, bn1_b, W2, b2,
           root, bias, bn_g, bn_b):
    src = edge_index[0]
    dst = edge_index[1]

    G, s, ea_packed = _moments(edge_attr)
    mu = s[0] / E
    cov = G / E - jnp.outer(mu, mu)
    var_h = jnp.sum(W1 * (cov @ W1), axis=0)
    mean_h = mu @ W1 + b1
    scale = bn1_g / jnp.sqrt(var_h + EPS)
    w1f = W1 * scale[None, :]
    c1 = (b1 - mean_h) * scale + bn1_b
    c1 = c1.reshape(1, HID)

    w2g = W2.reshape(HID, NODE_F, OUT1)
    w2p = jnp.zeros((HID, NODE_F, OUTP), _f32).at[:, :, :OUT1].set(w2g)
    w2p = w2p.reshape(HID, NODE_F * OUTP)
    b2g = b2.reshape(NODE_F, OUT1)
    b2p = jnp.zeros((NODE_F, OUTP), _f32).at[:, :OUT1].set(b2g)
    b2p = b2p.reshape(1, NODE_F * OUTP)

    x_pad = jnp.concatenate([x, jnp.zeros((N, 8 - NODE_F), _f32)], axis=1)
    x_j = _sc_gather(x_pad, src)

    xj_packed = jnp.reshape(x_j, (E // 16, 128))
    w1big = jnp.kron(jnp.eye(8, dtype=_f32), w1f)
    c1big = jnp.tile(c1, (1, 8))
    msg_packed = _edge_msgs(ea_packed, xj_packed, w1big, c1big, w2p, b2p,
                            jnp.asarray(_XB))

    msg = jnp.reshape(msg_packed, (E, OUTP))
    zrows = jnp.zeros((ROWS_T, OUTP), _f32)
    aggr = _sc_scatter(msg, dst, zrows)

    root32 = jnp.concatenate([root, jnp.eye(NODE_F, dtype=_f32)], axis=1)
    bias32 = jnp.concatenate([bias, jnp.zeros((NODE_F,), _f32)])
    bias32 = bias32.reshape(1, OUTP)
    out32, st = _combine(aggr, x, root32, bias32)

    mean = st[0] / N
    var = st[1] / N - mean * mean
    inv = jnp.concatenate([bn_g, jnp.ones((NODE_F,), _f32)])
    inv = inv / jnp.sqrt(jnp.concatenate(
        [var[:OUT1], jnp.ones((NODE_F,), _f32)]) + jnp.concatenate(
        [jnp.full((OUT1,), EPS, _f32), jnp.zeros((NODE_F,), _f32)]))
    sh = jnp.concatenate([bn_b, jnp.zeros((NODE_F,), _f32)]) - mean * inv
    sh = sh.at[OUT1:].set(0.0)

    return _epilogue(out32, inv.reshape(1, OUTP), sh.reshape(1, OUTP))


# speedup vs baseline: 2.2132x; 1.1278x over previous
import functools

import jax
import jax.numpy as jnp
import numpy as np
from jax import lax
from jax.experimental import pallas as pl
from jax.experimental.pallas import tpu as pltpu
from jax.experimental.pallas import tpu_sc as plsc

N = 50000
E = 800000
NODE_F = 4
EDGE_F = 16
HID = 128
OUT1 = 28
OUTP = 32
EPS = 1e-5

NC = 2
NS = 16
NW = NC * NS
CHUNK = 128
NCHUNKS = E // CHUNK
BASE_CH = NCHUNKS // NW
EXTRA = NCHUNKS - BASE_CH * NW
ROWS_PER_TILE = 3136
NPAD = NS * ROWS_PER_TILE

BLK_A = 8000
BLK_B = 6400
BLK_N = 5000

_f32 = jnp.float32

_XB = np.zeros((128, 16 * 128), np.float32)
for _m in range(16):
    for _i in range(NODE_F):
        _XB[8 * _m + _i, 128 * _m + 32 * _i:128 * _m + 32 * (_i + 1)] = 1.0

_GS = np.zeros((4, 128, 128), np.float32)
for _u in range(4):
    for _i2 in range(4):
        for _o in range(32):
            _GS[_u, 32 * _i2 + _o, 32 * _u + _o] = 1.0


def _moments_body(ea_ref, g_ref, s_ref, pk_ref, acc_g, acc_s):
    i = pl.program_id(0)

    @pl.when(i == 0)
    def _init():
        acc_g[...] = jnp.zeros_like(acc_g)
        acc_s[...] = jnp.zeros_like(acc_s)

    blk = ea_ref[...]
    acc_g[...] += lax.dot_general(blk, blk, (((0,), (0,)), ((), ())),
                                  preferred_element_type=_f32)
    acc_s[...] += jnp.sum(blk, axis=0, keepdims=True)
    blk3 = blk.reshape(BLK_A // 8, 8, EDGE_F)
    pk_ref[...] = jnp.concatenate(
        [blk3[:, j, :] for j in range(8)], axis=1)

    @pl.when(i == pl.num_programs(0) - 1)
    def _emit():
        g_ref[...] = acc_g[...]
        s_ref[...] = acc_s[...]


def _moments(edge_attr):
    return pl.pallas_call(
        _moments_body,
        grid=(E // BLK_A,),
        in_specs=[pl.BlockSpec((BLK_A, EDGE_F), lambda i: (i, 0))],
        out_specs=[pl.BlockSpec((EDGE_F, EDGE_F), lambda i: (0, 0)),
                   pl.BlockSpec((1, EDGE_F), lambda i: (0, 0)),
                   pl.BlockSpec((BLK_A // 8, 128), lambda i: (i, 0))],
        out_shape=[jax.ShapeDtypeStruct((EDGE_F, EDGE_F), _f32),
                   jax.ShapeDtypeStruct((1, EDGE_F), _f32),
                   jax.ShapeDtypeStruct((E // 8, 128), _f32)],
        scratch_shapes=[pltpu.VMEM((EDGE_F, EDGE_F), _f32),
                        pltpu.VMEM((1, EDGE_F), _f32)],
    )(edge_attr)


def _worker_partition(wid):
    n = BASE_CH + jnp.where(wid < EXTRA, 1, 0)
    start = wid * BASE_CH + jnp.minimum(wid, EXTRA)
    return start, n


SUP = 8


def _sc_gather(x_pad, src):
    mesh = plsc.VectorSubcoreMesh(core_axis_name="c", subcore_axis_name="s")

    @functools.partial(
        pl.kernel,
        out_type=jax.ShapeDtypeStruct((E, 8), _f32),
        mesh=mesh,
        compiler_params=pltpu.CompilerParams(use_tc_tiling_on_sc=False),
        scratch_types=[pltpu.VMEM((SUP * CHUNK,), jnp.int32),
                       pltpu.VMEM((SUP * CHUNK, 8), _f32),
                       pltpu.SemaphoreType.DMA],
    )
    def gk(x_hbm, src_hbm, out_hbm, idx_v, rows_v, sem):
        wid = lax.axis_index("s") * NC + lax.axis_index("c")
        start, n = _worker_partition(wid)

        def super_body(t, carry):
            off = (start + t * SUP) * CHUNK
            pltpu.sync_copy(src_hbm.at[pl.ds(off, SUP * CHUNK)], idx_v)
            copies = [
                pltpu.async_copy(x_hbm.at[idx_v.at[pl.ds(k * CHUNK, CHUNK)]],
                                 rows_v.at[pl.ds(k * CHUNK, CHUNK)], sem)
                for k in range(SUP)]
            for c in copies:
                c.wait()
            pltpu.sync_copy(rows_v, out_hbm.at[pl.ds(off, SUP * CHUNK)])
            return carry

        def tail_body(j, carry):
            off = (start + j) * CHUNK
            pltpu.sync_copy(src_hbm.at[pl.ds(off, CHUNK)],
                            idx_v.at[pl.ds(0, CHUNK)])
            pltpu.async_copy(x_hbm.at[idx_v.at[pl.ds(0, CHUNK)]],
                             rows_v.at[pl.ds(0, CHUNK)], sem).wait()
            pltpu.sync_copy(rows_v.at[pl.ds(0, CHUNK)],
                            out_hbm.at[pl.ds(off, CHUNK)])
            return carry

        nsup = n // SUP
        lax.fori_loop(0, nsup, super_body, 0)
        lax.fori_loop(nsup * SUP, n, tail_body, 0)

    return gk(x_pad, src)


def _edge_body(ea_ref, xj_ref, w1_ref, c1_ref, w2_ref, b2_ref, xb_ref,
               gsp_ref, msg_ref):
    h1p = jnp.maximum(
        jnp.dot(ea_ref[...], w1_ref[...], preferred_element_type=_f32)
        + c1_ref[...], 0.0)
    h1 = h1p.reshape(BLK_B, HID)
    h2 = jnp.maximum(
        jnp.dot(h1, w2_ref[...], preferred_element_type=_f32)
        + b2_ref[...], 0.0)
    xbc = jnp.dot(xj_ref[...], xb_ref[...],
                  preferred_element_type=_f32).reshape(BLK_B, 128)
    t = (xbc * h2).reshape(BLK_B // 4, 4, 128)
    msg_ref[...] = sum(
        jnp.dot(t[:, u, :], gsp_ref[u], preferred_element_type=_f32)
        for u in range(4))


def _edge_msgs(ea_packed, xj_packed, w1big, c1big, w2p, b2p, xbig, gsp):
    return pl.pallas_call(
        _edge_body,
        grid=(E // BLK_B,),
        in_specs=[pl.BlockSpec((BLK_B // 8, 128), lambda i: (i, 0)),
                  pl.BlockSpec((BLK_B // 16, 128), lambda i: (i, 0)),
                  pl.BlockSpec((HID, 8 * HID), lambda i: (0, 0)),
                  pl.BlockSpec((1, 8 * HID), lambda i: (0, 0)),
                  pl.BlockSpec((HID, HID), lambda i: (0, 0)),
                  pl.BlockSpec((1, HID), lambda i: (0, 0)),
                  pl.BlockSpec((128, 16 * 128), lambda i: (0, 0)),
                  pl.BlockSpec((4, 128, 128), lambda i: (0, 0, 0))],
        out_specs=pl.BlockSpec((BLK_B // 4, 128), lambda i: (i, 0)),
        out_shape=jax.ShapeDtypeStruct((E // 4, 128), _f32),
    )(ea_packed, xj_packed, w1big, c1big, w2p, b2p, xbig, gsp)


H_HALF = NPAD // 2
ROWS_T = H_HALF // NS
ACC_ROWS = H_HALF + NS
CH_BASE = NCHUNKS // NS
CH_EXTRA = NCHUNKS - CH_BASE * NS


def _sc_scatter(msg, dst, zrows):
    mesh = plsc.VectorSubcoreMesh(core_axis_name="c", subcore_axis_name="s")

    @functools.partial(
        pl.kernel,
        out_type=jax.ShapeDtypeStruct((NPAD, OUTP), _f32),
        mesh=mesh,
        compiler_params=pltpu.CompilerParams(use_tc_tiling_on_sc=False),
        scratch_types=[pltpu.VMEM((SUP * CHUNK,), jnp.int32),
                       pltpu.VMEM((SUP, CHUNK), jnp.int32),
                       pltpu.VMEM((SUP * CHUNK, OUTP), _f32),
                       pltpu.VMEM_SHARED((ACC_ROWS, OUTP), _f32),
                       pltpu.SemaphoreType.DMA],
    )
    def sk(msg_hbm, dst_hbm, z_hbm, out_hbm, idx_v, loc_v, rows_v,
           accum, sem):
        cid = lax.axis_index("c")
        sid = lax.axis_index("s")
        base_node = cid * H_HALF
        pltpu.sync_copy(z_hbm, accum.at[pl.ds(sid * ROWS_T, ROWS_T)])
        plsc.subcore_barrier()

        n = CH_BASE + jnp.where(sid < CH_EXTRA, 1, 0)
        start = sid * CH_BASE + jnp.minimum(sid, CH_EXTRA)

        def filter_chunk(k):
            for m in range(CHUNK // 16):
                v = idx_v[pl.ds(k * CHUNK + m * 16, 16)] - base_node
                ok = (v >= 0) & (v < H_HALF)
                loc_v[k, pl.ds(m * 16, 16)] = jnp.where(
                    ok, v, jnp.int32(H_HALF))

        def super_body(t, carry):
            off = (start + t * SUP) * CHUNK
            pltpu.sync_copy(dst_hbm.at[pl.ds(off, SUP * CHUNK)], idx_v)
            pltpu.sync_copy(msg_hbm.at[pl.ds(off, SUP * CHUNK)], rows_v)
            copies = []
            for k in range(SUP):
                filter_chunk(k)
                copies.append(pltpu.async_copy(
                    rows_v.at[pl.ds(k * CHUNK, CHUNK)],
                    accum.at[loc_v.at[k]], sem, add=True))
            for c in copies:
                c.wait()
            return carry

        def tail_body(j, carry):
            off = (start + j) * CHUNK
            pltpu.sync_copy(dst_hbm.at[pl.ds(off, CHUNK)],
                            idx_v.at[pl.ds(0, CHUNK)])
            pltpu.sync_copy(msg_hbm.at[pl.ds(off, CHUNK)],
                            rows_v.at[pl.ds(0, CHUNK)])
            filter_chunk(0)
            pltpu.sync_copy(rows_v.at[pl.ds(0, CHUNK)],
                            accum.at[loc_v.at[0]], add=True)
            return carry

        nsup = n // SUP
        lax.fori_loop(0, nsup, super_body, 0)
        lax.fori_loop(nsup * SUP, n, tail_body, 0)
        plsc.subcore_barrier()
        pltpu.sync_copy(accum.at[pl.ds(sid * ROWS_T, ROWS_T)],
                        out_hbm.at[pl.ds(base_node + sid * ROWS_T, ROWS_T)])

    return sk(msg, dst, zrows)


def _c1_body(p_ref, x_ref, r_ref, b_ref, out_ref, st_ref, acc):
    i = pl.program_id(0)

    @pl.when(i == 0)
    def _init():
        acc[...] = jnp.zeros_like(acc)

    blk = (p_ref[...]
           + jnp.dot(x_ref[...], r_ref[...], preferred_element_type=_f32)
           + b_ref[...])
    out_ref[...] = blk
    acc[0:1, :] += jnp.sum(blk, axis=0, keepdims=True)
    acc[1:2, :] += jnp.sum(blk * blk, axis=0, keepdims=True)

    @pl.when(i == pl.num_programs(0) - 1)
    def _emit():
        st_ref[...] = acc[...]


def _combine(p, x, root32, bias32):
    return pl.pallas_call(
        _c1_body,
        grid=(N // BLK_N,),
        in_specs=[pl.BlockSpec((BLK_N, OUTP), lambda i: (i, 0)),
                  pl.BlockSpec((BLK_N, NODE_F), lambda i: (i, 0)),
                  pl.BlockSpec((NODE_F, OUTP), lambda i: (0, 0)),
                  pl.BlockSpec((1, OUTP), lambda i: (0, 0))],
        out_specs=[pl.BlockSpec((BLK_N, OUTP), lambda i: (i, 0)),
                   pl.BlockSpec((2, OUTP), lambda i: (0, 0))],
        out_shape=[jax.ShapeDtypeStruct((N, OUTP), _f32),
                   jax.ShapeDtypeStruct((2, OUTP), _f32)],
        scratch_shapes=[pltpu.VMEM((2, OUTP), _f32)],
    )(p, x, root32, bias32)


def _c2_body(o_ref, inv_ref, sh_ref, y_ref):
    y_ref[...] = jnp.maximum(o_ref[...] * inv_ref[...] + sh_ref[...], 0.0)


def _epilogue(out32, inv32, sh32):
    return pl.pallas_call(
        _c2_body,
        grid=(N // BLK_N,),
        in_specs=[pl.BlockSpec((BLK_N, OUTP), lambda i: (i, 0)),
                  pl.BlockSpec((1, OUTP), lambda i: (0, 0)),
                  pl.BlockSpec((1, OUTP), lambda i: (0, 0))],
        out_specs=pl.BlockSpec((BLK_N, OUTP), lambda i: (i, 0)),
        out_shape=jax.ShapeDtypeStruct((N, OUTP), _f32),
    )(out32, inv32, sh32)


def kernel(x, edge_index, edge_attr, W1, b1, bn1_g, bn1_b, W2, b2,
           root, bias, bn_g, bn_b):
    src = edge_index[0]
    dst = edge_index[1]

    G, s, ea_packed = _moments(edge_attr)
    mu = s[0] / E
    cov = G / E - jnp.outer(mu, mu)
    var_h = jnp.sum(W1 * (cov @ W1), axis=0)
    mean_h = mu @ W1 + b1
    scale = bn1_g / jnp.sqrt(var_h + EPS)
    w1f = W1 * scale[None, :]
    c1 = (b1 - mean_h) * scale + bn1_b
    c1 = c1.reshape(1, HID)

    w2g = W2.reshape(HID, NODE_F, OUT1)
    w2p = jnp.zeros((HID, NODE_F, OUTP), _f32).at[:, :, :OUT1].set(w2g)
    w2p = w2p.reshape(HID, NODE_F * OUTP)
    b2g = b2.reshape(NODE_F, OUT1)
    b2p = jnp.zeros((NODE_F, OUTP), _f32).at[:, :OUT1].set(b2g)
    b2p = b2p.reshape(1, NODE_F * OUTP)

    x_pad = jnp.concatenate([x, jnp.zeros((N, 8 - NODE_F), _f32)], axis=1)
    x_j = _sc_gather(x_pad, src)

    xj_packed = jnp.reshape(x_j, (E // 16, 128))
    w1big = jnp.kron(jnp.eye(8, dtype=_f32), w1f)
    c1big = jnp.tile(c1, (1, 8))
    msg_packed = _edge_msgs(ea_packed, xj_packed, w1big, c1big, w2p, b2p,
                            jnp.asarray(_XB), jnp.asarray(_GS))

    msg = jnp.reshape(msg_packed, (E, OUTP))
    zrows = jnp.zeros((ROWS_T, OUTP), _f32)
    aggr = _sc_scatter(msg, dst, zrows)

    root32 = jnp.concatenate([root, jnp.eye(NODE_F, dtype=_f32)], axis=1)
    bias32 = jnp.concatenate([bias, jnp.zeros((NODE_F,), _f32)])
    bias32 = bias32.reshape(1, OUTP)
    out32, st = _combine(aggr, x, root32, bias32)

    mean = st[0] / N
    var = st[1] / N - mean * mean
    inv = jnp.concatenate([bn_g, jnp.ones((NODE_F,), _f32)])
    inv = inv / jnp.sqrt(jnp.concatenate(
        [var[:OUT1], jnp.ones((NODE_F,), _f32)]) + jnp.concatenate(
        [jnp.full((OUT1,), EPS, _f32), jnp.zeros((NODE_F,), _f32)]))
    sh = jnp.concatenate([bn_b, jnp.zeros((NODE_F,), _f32)]) - mean * inv
    sh = sh.at[OUT1:].set(0.0)

    return _epilogue(out32, inv.reshape(1, OUTP), sh.reshape(1, OUTP))
